# baseline (speedup 1.0000x reference)
import jax
import jax.numpy as jnp
from jax import lax
from jax.experimental import pallas as pl
from jax.experimental.pallas import tpu as pltpu
from jax.experimental.pallas import tpu_sc as plsc

N_TOK = 2048
C_DIM = 1024
N_EXP = 8
F_DIM = 4096
TOPK = 2
TILE = 256
MAXT = 23
PROWS = MAXT * TILE
FBLK = 1024
NFB = F_DIM // FBLK
CH = 64


def _router_body(x_ref, w_ref, b_ref, pos_ref, wts_ref, meta_ref, bal_ref):
    f32 = jnp.float32
    xv = x_ref[...]
    logits = jnp.dot(xv, w_ref[...], preferred_element_type=f32) + b_ref[...]
    m = jnp.max(logits, axis=1, keepdims=True)
    ex = jnp.exp(logits - m)
    probs = ex / jnp.sum(ex, axis=1, keepdims=True)
    mean_p = jnp.sum(probs, axis=0, keepdims=True) * (1.0 / N_TOK)
    bal_ref[...] = jnp.sum(mean_p * mean_p, axis=1, keepdims=True)
    lane = lax.broadcasted_iota(jnp.int32, (N_TOK, N_EXP), 1)
    i1 = jnp.min(jnp.where(logits == m, lane, N_EXP), axis=1, keepdims=True)
    masked = jnp.where(lane == i1, -jnp.inf, logits)
    m2 = jnp.max(masked, axis=1, keepdims=True)
    i2 = jnp.min(jnp.where(masked == m2, lane, N_EXP), axis=1, keepdims=True)
    v1 = jnp.sum(jnp.where(lane == i1, probs, 0.0), axis=1, keepdims=True)
    v2 = jnp.sum(jnp.where(lane == i2, probs, 0.0), axis=1, keepdims=True)
    wts_ref[:, 0:1] = v1
    wts_ref[:, 1:2] = v2
    I1 = (lane == i1).astype(f32)
    I2 = (lane == i2).astype(f32)
    counts = jnp.sum(I1 + I2, axis=0, keepdims=True)
    nt = jnp.floor((counts + (TILE - 1)) * (1.0 / TILE))
    er = lax.broadcasted_iota(jnp.int32, (N_EXP, N_EXP), 0)
    ec = lax.broadcasted_iota(jnp.int32, (N_EXP, N_EXP), 1)
    strict = (er < ec).astype(f32)
    tiles_before = jnp.dot(nt, strict, preferred_element_type=f32)
    start = tiles_before * float(TILE)
    tiles_incl = tiles_before + nt
    ident = (er == ec).astype(f32)
    ti_col = jnp.sum(ident * tiles_incl, axis=1, keepdims=True)
    lane32 = lax.broadcasted_iota(jnp.int32, (1, 32), 1).astype(f32)
    te = jnp.sum((ti_col <= lane32).astype(f32), axis=0, keepdims=True)
    te = jnp.minimum(te, float(N_EXP - 1))
    used = tiles_incl[:, N_EXP - 1:N_EXP]
    meta_ref[...] = jnp.where(lane32 == 31.0, used, te).astype(jnp.int32)
    L = (lax.broadcasted_iota(jnp.int32, (128, 128), 0)
         >= lax.broadcasted_iota(jnp.int32, (128, 128), 1)).astype(f32)
    lane8 = lax.broadcasted_iota(jnp.int32, (128, N_EXP), 1)
    carry = jnp.zeros((1, N_EXP), f32)
    for blk in range(32):
        k, rb = divmod(blk, 16)
        r0 = rb * 128
        ei = (i1 if k == 0 else i2)[r0:r0 + 128, :]
        Ic = (lane8 == ei).astype(f32)
        incl = jnp.dot(L, Ic, preferred_element_type=f32)
        rank = carry + incl - Ic
        posb = jnp.sum(Ic * (start + rank), axis=1, keepdims=True)
        pos_ref[r0:r0 + 128, k:k + 1] = posb.astype(jnp.int32)
        carry = carry + incl[127:128, :]


def _router_call(x_flat, router_W, router_b):
    return pl.pallas_call(
        _router_body,
        out_shape=(
            jax.ShapeDtypeStruct((N_TOK, TOPK), jnp.int32),
            jax.ShapeDtypeStruct((N_TOK, TOPK), jnp.float32),
            jax.ShapeDtypeStruct((1, 32), jnp.int32),
            jax.ShapeDtypeStruct((1, 1), jnp.float32),
        ),
    )(x_flat, router_W, router_b.reshape(1, N_EXP))


SCH = 32


def _sc_scatter_body(x_hbm, pos_hbm, wsrc_hbm, xg_hbm, ws_hbm,
                     xb0, xb1, wb0, wb1, ib0, ib1, s0, s1, t0, t1):
    wid = lax.axis_index("s") * 2 + lax.axis_index("c")
    k = wid // 16
    base = (wid % 16) * 128
    xb = (xb0, xb1)
    wb = (wb0, wb1)
    ib = (ib0, ib1)
    xsem = (s0, s1)
    wsem = (t0, t1)
    pend = [None, None]
    for c in range(4):
        b = c % 2
        if pend[b] is not None:
            pend[b][0].wait()
            pend[b][1].wait()
        tb = base + c * SCH
        pltpu.sync_copy(pos_hbm.at[pl.ds(k * N_TOK + tb, SCH)], ib[b])
        pltpu.sync_copy(wsrc_hbm.at[pl.ds(k * N_TOK + tb, SCH)], wb[b])
        pltpu.sync_copy(x_hbm.at[pl.ds(tb, SCH)], xb[b])
        pend[b] = (pltpu.async_copy(xb[b], xg_hbm.at[ib[b]], xsem[b]),
                   pltpu.async_copy(wb[b], ws_hbm.at[ib[b]], wsem[b]))
    for b in range(2):
        pend[b][0].wait()
        pend[b][1].wait()


def _sc_scatter(x_flat, pos_flat, wsrc):
    mesh = plsc.VectorSubcoreMesh(core_axis_name="c", subcore_axis_name="s")
    fn = pl.kernel(
        _sc_scatter_body, mesh=mesh,
        out_type=(jax.ShapeDtypeStruct((PROWS, C_DIM), jnp.float32),
                  jax.ShapeDtypeStruct((PROWS, 128), jnp.float32)),
        scratch_types=[pltpu.VMEM((SCH, C_DIM), jnp.float32),
                       pltpu.VMEM((SCH, C_DIM), jnp.float32),
                       pltpu.VMEM((SCH, 128), jnp.float32),
                       pltpu.VMEM((SCH, 128), jnp.float32),
                       pltpu.VMEM((SCH,), jnp.int32),
                       pltpu.VMEM((SCH,), jnp.int32),
                       pltpu.SemaphoreType.DMA,
                       pltpu.SemaphoreType.DMA,
                       pltpu.SemaphoreType.DMA,
                       pltpu.SemaphoreType.DMA],
    )
    return fn(x_flat, pos_flat, wsrc)


def _ffn1_body(meta_ref, xg_ref, w1_ref, b1_ref, h_ref):
    t = pl.program_id(0)
    used = meta_ref[31]

    @pl.when(t < used)
    def _():
        h = jnp.dot(xg_ref[...].astype(jnp.bfloat16), w1_ref[0].astype(jnp.bfloat16),
                    preferred_element_type=jnp.float32) + b1_ref[0]
        h = 0.5 * h * (1.0 + lax.erf(h * 0.7071067811865476))
        h_ref[...] = h.astype(jnp.bfloat16)


def _ffn2_body(meta_ref, h_ref, w2_ref, b2_ref, ws_ref, out_ref):
    t = pl.program_id(0)
    used = meta_ref[31]

    @pl.when(t < used)
    def _():
        contrib = jnp.dot(h_ref[...], w2_ref[0].astype(jnp.bfloat16),
                          preferred_element_type=jnp.float32)
        out_ref[...] = (contrib + b2_ref[0]) * ws_ref[:, 0:1]


def _ffn_call(meta, xg, W1, b1, W2, b2, wS):
    def live_in(t, m):
        return jnp.where(t < m[31], t, 0)

    def live_out(t, m):
        return jnp.where(t < m[31], t, MAXT)

    grid_spec1 = pltpu.PrefetchScalarGridSpec(
        num_scalar_prefetch=1,
        grid=(MAXT,),
        in_specs=[
            pl.BlockSpec((TILE, C_DIM), lambda t, m: (live_in(t, m), 0)),
            pl.BlockSpec((1, C_DIM, F_DIM), lambda t, m: (m[t], 0, 0)),
            pl.BlockSpec((1, 1, F_DIM), lambda t, m: (m[t], 0, 0)),
        ],
        out_specs=pl.BlockSpec((TILE, F_DIM), lambda t, m: (live_out(t, m), 0)),
    )
    h = pl.pallas_call(
        _ffn1_body, grid_spec=grid_spec1,
        out_shape=jax.ShapeDtypeStruct((PROWS + TILE, F_DIM), jnp.bfloat16),
        compiler_params=pltpu.CompilerParams(vmem_limit_bytes=60 * 1024 * 1024),
    )(meta, xg, W1, b1)
    grid_spec2 = pltpu.PrefetchScalarGridSpec(
        num_scalar_prefetch=1,
        grid=(MAXT,),
        in_specs=[
            pl.BlockSpec((TILE, F_DIM), lambda t, m: (live_in(t, m), 0)),
            pl.BlockSpec((1, F_DIM, C_DIM), lambda t, m: (m[t], 0, 0)),
            pl.BlockSpec((1, 1, C_DIM), lambda t, m: (m[t], 0, 0)),
            pl.BlockSpec((TILE, 128), lambda t, m: (live_in(t, m), 0)),
        ],
        out_specs=pl.BlockSpec((TILE, C_DIM), lambda t, m: (live_out(t, m), 0)),
    )
    return pl.pallas_call(
        _ffn2_body, grid_spec=grid_spec2,
        out_shape=jax.ShapeDtypeStruct((PROWS + TILE, C_DIM), jnp.float32),
        compiler_params=pltpu.CompilerParams(vmem_limit_bytes=60 * 1024 * 1024),
    )(meta, h, W2, b2, wS)


GCH = 32


def _sc_gather_body(o_hbm, pos_hbm, out_hbm, r0, r1, idx0, idx1, sem0, sem1):
    wid = lax.axis_index("s") * 2 + lax.axis_index("c")
    tb = wid * CH
    for sub in range(CH // GCH):
        cb = tb + sub * GCH
        pltpu.sync_copy(pos_hbm.at[pl.ds(cb, GCH)], idx0)
        pltpu.sync_copy(pos_hbm.at[pl.ds(N_TOK + cb, GCH)], idx1)
        c0 = pltpu.async_copy(o_hbm.at[idx0], r0, sem0)
        c1 = pltpu.async_copy(o_hbm.at[idx1], r1, sem1)
        c0.wait()
        c1.wait()

        def row_add(j, _):
            for c in range(C_DIM // 16):
                sl = (j, pl.ds(c * 16, 16))
                r0[sl] = r0[sl] + r1[sl]
            return 0

        lax.fori_loop(0, GCH, row_add, 0)
        pltpu.sync_copy(r0, out_hbm.at[pl.ds(cb, GCH)])


def _sc_gather_add(o, pos_flat):
    mesh = plsc.VectorSubcoreMesh(core_axis_name="c", subcore_axis_name="s")
    fn = pl.kernel(
        _sc_gather_body, mesh=mesh,
        out_type=jax.ShapeDtypeStruct((N_TOK, C_DIM), jnp.float32),
        scratch_types=[pltpu.VMEM((GCH, C_DIM), jnp.float32),
                       pltpu.VMEM((GCH, C_DIM), jnp.float32),
                       pltpu.VMEM((GCH,), jnp.int32),
                       pltpu.VMEM((GCH,), jnp.int32),
                       pltpu.SemaphoreType.DMA,
                       pltpu.SemaphoreType.DMA],
    )
    return fn(o, pos_flat)


def kernel(x, router_W, router_b, W1, b1, W2, b2):
    B, T, C = x.shape
    x_flat = x.reshape(T, C)
    pos, wts, meta, bal = _router_call(x_flat, router_W, router_b)
    pos_flat = pos.T.reshape(TOPK * N_TOK)
    wsrc = jnp.broadcast_to(wts.T.reshape(TOPK * N_TOK, 1), (TOPK * N_TOK, 128))
    xg, wS = _sc_scatter(x_flat, pos_flat, wsrc)
    o = _ffn_call(meta.reshape(32), xg,
                  W1, b1.reshape(N_EXP, 1, F_DIM),
                  W2, b2.reshape(N_EXP, 1, C_DIM), wS)
    out = _sc_gather_add(o, pos_flat)
    return out.reshape(B, T, C), bal.reshape(())


# speedup vs baseline: 1.0642x; 1.0642x over previous
import jax
import jax.numpy as jnp
from jax import lax
from jax.experimental import pallas as pl
from jax.experimental.pallas import tpu as pltpu
from jax.experimental.pallas import tpu_sc as plsc

N_TOK = 2048
C_DIM = 1024
N_EXP = 8
F_DIM = 4096
TOPK = 2
TILE = 512
MAXT = 15
PROWS = MAXT * TILE
FBLK = 1024
NFB = F_DIM // FBLK
CH = 64


def _router_body(x_ref, w_ref, b_ref, pos_ref, wts_ref, meta_ref, bal_ref):
    f32 = jnp.float32
    xv = x_ref[...]
    logits = jnp.dot(xv, w_ref[...], preferred_element_type=f32) + b_ref[...]
    m = jnp.max(logits, axis=1, keepdims=True)
    ex = jnp.exp(logits - m)
    probs = ex / jnp.sum(ex, axis=1, keepdims=True)
    mean_p = jnp.sum(probs, axis=0, keepdims=True) * (1.0 / N_TOK)
    bal_ref[...] = jnp.sum(mean_p * mean_p, axis=1, keepdims=True)
    lane = lax.broadcasted_iota(jnp.int32, (N_TOK, N_EXP), 1)
    i1 = jnp.min(jnp.where(logits == m, lane, N_EXP), axis=1, keepdims=True)
    masked = jnp.where(lane == i1, -jnp.inf, logits)
    m2 = jnp.max(masked, axis=1, keepdims=True)
    i2 = jnp.min(jnp.where(masked == m2, lane, N_EXP), axis=1, keepdims=True)
    v1 = jnp.sum(jnp.where(lane == i1, probs, 0.0), axis=1, keepdims=True)
    v2 = jnp.sum(jnp.where(lane == i2, probs, 0.0), axis=1, keepdims=True)
    wts_ref[:, 0:1] = v1
    wts_ref[:, 1:2] = v2
    I1 = (lane == i1).astype(f32)
    I2 = (lane == i2).astype(f32)
    counts = jnp.sum(I1 + I2, axis=0, keepdims=True)
    nt = jnp.floor((counts + (TILE - 1)) * (1.0 / TILE))
    er = lax.broadcasted_iota(jnp.int32, (N_EXP, N_EXP), 0)
    ec = lax.broadcasted_iota(jnp.int32, (N_EXP, N_EXP), 1)
    strict = (er < ec).astype(f32)
    tiles_before = jnp.dot(nt, strict, preferred_element_type=f32)
    start = tiles_before * float(TILE)
    tiles_incl = tiles_before + nt
    ident = (er == ec).astype(f32)
    ti_col = jnp.sum(ident * tiles_incl, axis=1, keepdims=True)
    lane32 = lax.broadcasted_iota(jnp.int32, (1, 32), 1).astype(f32)
    te = jnp.sum((ti_col <= lane32).astype(f32), axis=0, keepdims=True)
    te = jnp.minimum(te, float(N_EXP - 1))
    used = tiles_incl[:, N_EXP - 1:N_EXP]
    meta_ref[...] = jnp.where(lane32 == 31.0, used, te).astype(jnp.int32)
    L = (lax.broadcasted_iota(jnp.int32, (128, 128), 0)
         >= lax.broadcasted_iota(jnp.int32, (128, 128), 1)).astype(f32)
    lane8 = lax.broadcasted_iota(jnp.int32, (128, N_EXP), 1)
    carry = jnp.zeros((1, N_EXP), f32)
    for blk in range(32):
        k, rb = divmod(blk, 16)
        r0 = rb * 128
        ei = (i1 if k == 0 else i2)[r0:r0 + 128, :]
        Ic = (lane8 == ei).astype(f32)
        incl = jnp.dot(L, Ic, preferred_element_type=f32)
        rank = carry + incl - Ic
        posb = jnp.sum(Ic * (start + rank), axis=1, keepdims=True)
        pos_ref[r0:r0 + 128, k:k + 1] = posb.astype(jnp.int32)
        carry = carry + incl[127:128, :]


def _router_call(x_flat, router_W, router_b):
    return pl.pallas_call(
        _router_body,
        out_shape=(
            jax.ShapeDtypeStruct((N_TOK, TOPK), jnp.int32),
            jax.ShapeDtypeStruct((N_TOK, TOPK), jnp.float32),
            jax.ShapeDtypeStruct((1, 32), jnp.int32),
            jax.ShapeDtypeStruct((1, 1), jnp.float32),
        ),
    )(x_flat, router_W, router_b.reshape(1, N_EXP))


SCH = 32


def _sc_scatter_body(x_hbm, pos_hbm, wsrc_hbm, xg_hbm, ws_hbm,
                     xb0, xb1, wb0, wb1, ib0, ib1, s0, s1, t0, t1):
    wid = lax.axis_index("s") * 2 + lax.axis_index("c")
    k = wid // 16
    base = (wid % 16) * 128
    xb = (xb0, xb1)
    wb = (wb0, wb1)
    ib = (ib0, ib1)
    xsem = (s0, s1)
    wsem = (t0, t1)
    pend = [None, None]
    for c in range(4):
        b = c % 2
        if pend[b] is not None:
            pend[b][0].wait()
            pend[b][1].wait()
        tb = base + c * SCH
        pltpu.sync_copy(pos_hbm.at[pl.ds(k * N_TOK + tb, SCH)], ib[b])
        pltpu.sync_copy(wsrc_hbm.at[pl.ds(k * N_TOK + tb, SCH)], wb[b])
        pltpu.sync_copy(x_hbm.at[pl.ds(tb, SCH)], xb[b])
        pend[b] = (pltpu.async_copy(xb[b], xg_hbm.at[ib[b]], xsem[b]),
                   pltpu.async_copy(wb[b], ws_hbm.at[ib[b]], wsem[b]))
    for b in range(2):
        pend[b][0].wait()
        pend[b][1].wait()


def _sc_scatter(x_flat, pos_flat, wsrc):
    mesh = plsc.VectorSubcoreMesh(core_axis_name="c", subcore_axis_name="s")
    fn = pl.kernel(
        _sc_scatter_body, mesh=mesh,
        out_type=(jax.ShapeDtypeStruct((PROWS, C_DIM), jnp.float32),
                  jax.ShapeDtypeStruct((PROWS, 128), jnp.float32)),
        scratch_types=[pltpu.VMEM((SCH, C_DIM), jnp.float32),
                       pltpu.VMEM((SCH, C_DIM), jnp.float32),
                       pltpu.VMEM((SCH, 128), jnp.float32),
                       pltpu.VMEM((SCH, 128), jnp.float32),
                       pltpu.VMEM((SCH,), jnp.int32),
                       pltpu.VMEM((SCH,), jnp.int32),
                       pltpu.SemaphoreType.DMA,
                       pltpu.SemaphoreType.DMA,
                       pltpu.SemaphoreType.DMA,
                       pltpu.SemaphoreType.DMA],
    )
    return fn(x_flat, pos_flat, wsrc)


def _ffn1_body(meta_ref, xg_ref, w1_ref, b1_ref, h_ref):
    t = pl.program_id(0)
    used = meta_ref[31]

    @pl.when(t < used)
    def _():
        h = jnp.dot(xg_ref[...].astype(jnp.bfloat16), w1_ref[0].astype(jnp.bfloat16),
                    preferred_element_type=jnp.float32) + b1_ref[0]
        h = 0.5 * h * (1.0 + lax.erf(h * 0.7071067811865476))
        h_ref[...] = h.astype(jnp.bfloat16)


def _ffn2_body(meta_ref, h_ref, w2_ref, b2_ref, ws_ref, out_ref):
    t = pl.program_id(0)
    used = meta_ref[31]

    @pl.when(t < used)
    def _():
        contrib = jnp.dot(h_ref[...], w2_ref[0].astype(jnp.bfloat16),
                          preferred_element_type=jnp.float32)
        out_ref[...] = (contrib + b2_ref[0]) * ws_ref[:, 0:1]


def _ffn_call(meta, xg, W1, b1, W2, b2, wS):
    def live_in(t, m):
        return jnp.where(t < m[31], t, 0)

    def live_out(t, m):
        return jnp.where(t < m[31], t, MAXT)

    grid_spec1 = pltpu.PrefetchScalarGridSpec(
        num_scalar_prefetch=1,
        grid=(MAXT,),
        in_specs=[
            pl.BlockSpec((TILE, C_DIM), lambda t, m: (live_in(t, m), 0)),
            pl.BlockSpec((1, C_DIM, F_DIM), lambda t, m: (m[t], 0, 0)),
            pl.BlockSpec((1, 1, F_DIM), lambda t, m: (m[t], 0, 0)),
        ],
        out_specs=pl.BlockSpec((TILE, F_DIM), lambda t, m: (live_out(t, m), 0)),
    )
    h = pl.pallas_call(
        _ffn1_body, grid_spec=grid_spec1,
        out_shape=jax.ShapeDtypeStruct((PROWS + TILE, F_DIM), jnp.bfloat16),
        compiler_params=pltpu.CompilerParams(vmem_limit_bytes=60 * 1024 * 1024),
    )(meta, xg, W1, b1)
    grid_spec2 = pltpu.PrefetchScalarGridSpec(
        num_scalar_prefetch=1,
        grid=(MAXT,),
        in_specs=[
            pl.BlockSpec((TILE, F_DIM), lambda t, m: (live_in(t, m), 0)),
            pl.BlockSpec((1, F_DIM, C_DIM), lambda t, m: (m[t], 0, 0)),
            pl.BlockSpec((1, 1, C_DIM), lambda t, m: (m[t], 0, 0)),
            pl.BlockSpec((TILE, 128), lambda t, m: (live_in(t, m), 0)),
        ],
        out_specs=pl.BlockSpec((TILE, C_DIM), lambda t, m: (live_out(t, m), 0)),
    )
    return pl.pallas_call(
        _ffn2_body, grid_spec=grid_spec2,
        out_shape=jax.ShapeDtypeStruct((PROWS + TILE, C_DIM), jnp.float32),
        compiler_params=pltpu.CompilerParams(vmem_limit_bytes=60 * 1024 * 1024),
    )(meta, h, W2, b2, wS)


GCH = 32


def _sc_gather_body(o_hbm, pos_hbm, out_hbm, r0, r1, idx0, idx1, sem0, sem1):
    wid = lax.axis_index("s") * 2 + lax.axis_index("c")
    tb = wid * CH
    for sub in range(CH // GCH):
        cb = tb + sub * GCH
        pltpu.sync_copy(pos_hbm.at[pl.ds(cb, GCH)], idx0)
        pltpu.sync_copy(pos_hbm.at[pl.ds(N_TOK + cb, GCH)], idx1)
        c0 = pltpu.async_copy(o_hbm.at[idx0], r0, sem0)
        c1 = pltpu.async_copy(o_hbm.at[idx1], r1, sem1)
        c0.wait()
        c1.wait()

        def row_add(j, _):
            for c in range(C_DIM // 16):
                sl = (j, pl.ds(c * 16, 16))
                r0[sl] = r0[sl] + r1[sl]
            return 0

        lax.fori_loop(0, GCH, row_add, 0)
        pltpu.sync_copy(r0, out_hbm.at[pl.ds(cb, GCH)])


def _sc_gather_add(o, pos_flat):
    mesh = plsc.VectorSubcoreMesh(core_axis_name="c", subcore_axis_name="s")
    fn = pl.kernel(
        _sc_gather_body, mesh=mesh,
        out_type=jax.ShapeDtypeStruct((N_TOK, C_DIM), jnp.float32),
        scratch_types=[pltpu.VMEM((GCH, C_DIM), jnp.float32),
                       pltpu.VMEM((GCH, C_DIM), jnp.float32),
                       pltpu.VMEM((GCH,), jnp.int32),
                       pltpu.VMEM((GCH,), jnp.int32),
                       pltpu.SemaphoreType.DMA,
                       pltpu.SemaphoreType.DMA],
    )
    return fn(o, pos_flat)


def kernel(x, router_W, router_b, W1, b1, W2, b2):
    B, T, C = x.shape
    x_flat = x.reshape(T, C)
    pos, wts, meta, bal = _router_call(x_flat, router_W, router_b)
    pos_flat = pos.T.reshape(TOPK * N_TOK)
    wsrc = jnp.broadcast_to(wts.T.reshape(TOPK * N_TOK, 1), (TOPK * N_TOK, 128))
    xg, wS = _sc_scatter(x_flat, pos_flat, wsrc)
    o = _ffn_call(meta.reshape(32), xg,
                  W1, b1.reshape(N_EXP, 1, F_DIM),
                  W2, b2.reshape(N_EXP, 1, C_DIM), wS)
    out = _sc_gather_add(o, pos_flat)
    return out.reshape(B, T, C), bal.reshape(())


# speedup vs baseline: 1.2575x; 1.1815x over previous
import jax
import jax.numpy as jnp
from jax import lax
from jax.experimental import pallas as pl
from jax.experimental.pallas import tpu as pltpu
from jax.experimental.pallas import tpu_sc as plsc

N_TOK = 2048
C_DIM = 1024
N_EXP = 8
F_DIM = 4096
TOPK = 2
TILE = 640
MAXT = 13
PROWS = MAXT * TILE
FBLK = 1024
NFB = F_DIM // FBLK
CH = 64


def _router_body(x_ref, w_ref, b_ref, pos_ref, wts_ref, meta_ref, bal_ref):
    f32 = jnp.float32
    xv = x_ref[...]
    logits = jnp.dot(xv, w_ref[...], preferred_element_type=f32) + b_ref[...]
    m = jnp.max(logits, axis=1, keepdims=True)
    ex = jnp.exp(logits - m)
    probs = ex / jnp.sum(ex, axis=1, keepdims=True)
    mean_p = jnp.sum(probs, axis=0, keepdims=True) * (1.0 / N_TOK)
    bal_ref[...] = jnp.sum(mean_p * mean_p, axis=1, keepdims=True)
    lane = lax.broadcasted_iota(jnp.int32, (N_TOK, N_EXP), 1)
    i1 = jnp.min(jnp.where(logits == m, lane, N_EXP), axis=1, keepdims=True)
    masked = jnp.where(lane == i1, -jnp.inf, logits)
    m2 = jnp.max(masked, axis=1, keepdims=True)
    i2 = jnp.min(jnp.where(masked == m2, lane, N_EXP), axis=1, keepdims=True)
    v1 = jnp.sum(jnp.where(lane == i1, probs, 0.0), axis=1, keepdims=True)
    v2 = jnp.sum(jnp.where(lane == i2, probs, 0.0), axis=1, keepdims=True)
    wts_ref[:, 0:1] = v1
    wts_ref[:, 1:2] = v2
    I1 = (lane == i1).astype(f32)
    I2 = (lane == i2).astype(f32)
    counts = jnp.sum(I1 + I2, axis=0, keepdims=True)
    nt = jnp.floor((counts + (TILE - 1)) * (1.0 / TILE))
    er = lax.broadcasted_iota(jnp.int32, (N_EXP, N_EXP), 0)
    ec = lax.broadcasted_iota(jnp.int32, (N_EXP, N_EXP), 1)
    strict = (er < ec).astype(f32)
    tiles_before = jnp.dot(nt, strict, preferred_element_type=f32)
    start = tiles_before * float(TILE)
    tiles_incl = tiles_before + nt
    ident = (er == ec).astype(f32)
    ti_col = jnp.sum(ident * tiles_incl, axis=1, keepdims=True)
    lane32 = lax.broadcasted_iota(jnp.int32, (1, 32), 1).astype(f32)
    te = jnp.sum((ti_col <= lane32).astype(f32), axis=0, keepdims=True)
    te = jnp.minimum(te, float(N_EXP - 1))
    used = tiles_incl[:, N_EXP - 1:N_EXP]
    meta_ref[...] = jnp.where(lane32 == 31.0, used, te).astype(jnp.int32)
    L = (lax.broadcasted_iota(jnp.int32, (128, 128), 0)
         >= lax.broadcasted_iota(jnp.int32, (128, 128), 1)).astype(f32)
    lane8 = lax.broadcasted_iota(jnp.int32, (128, N_EXP), 1)
    carry = jnp.zeros((1, N_EXP), f32)
    for blk in range(32):
        k, rb = divmod(blk, 16)
        r0 = rb * 128
        ei = (i1 if k == 0 else i2)[r0:r0 + 128, :]
        Ic = (lane8 == ei).astype(f32)
        incl = jnp.dot(L, Ic, preferred_element_type=f32)
        rank = carry + incl - Ic
        posb = jnp.sum(Ic * (start + rank), axis=1, keepdims=True)
        pos_ref[r0:r0 + 128, k:k + 1] = posb.astype(jnp.int32)
        carry = carry + incl[127:128, :]


def _router_call(x_flat, router_W, router_b):
    return pl.pallas_call(
        _router_body,
        out_shape=(
            jax.ShapeDtypeStruct((N_TOK, TOPK), jnp.int32),
            jax.ShapeDtypeStruct((N_TOK, TOPK), jnp.float32),
            jax.ShapeDtypeStruct((1, 32), jnp.int32),
            jax.ShapeDtypeStruct((1, 1), jnp.float32),
        ),
    )(x_flat, router_W, router_b.reshape(1, N_EXP))


SCH = 32


def _sc_scatter_body(x_hbm, pos_hbm, wsrc_hbm, xg_hbm, ws_hbm,
                     xb0, xb1, wb0, wb1, ib0, ib1, s0, s1, t0, t1):
    wid = lax.axis_index("s") * 2 + lax.axis_index("c")
    k = wid // 16
    base = (wid % 16) * 128
    xb = (xb0, xb1)
    wb = (wb0, wb1)
    ib = (ib0, ib1)
    xsem = (s0, s1)
    wsem = (t0, t1)
    pend = [None, None]
    for c in range(4):
        b = c % 2
        if pend[b] is not None:
            pend[b][0].wait()
            pend[b][1].wait()
        tb = base + c * SCH
        pltpu.sync_copy(pos_hbm.at[pl.ds(k * N_TOK + tb, SCH)], ib[b])
        pltpu.sync_copy(wsrc_hbm.at[pl.ds(k * N_TOK + tb, SCH)], wb[b])
        pltpu.sync_copy(x_hbm.at[pl.ds(tb, SCH)], xb[b])
        pend[b] = (pltpu.async_copy(xb[b], xg_hbm.at[ib[b]], xsem[b]),
                   pltpu.async_copy(wb[b], ws_hbm.at[ib[b]], wsem[b]))
    for b in range(2):
        pend[b][0].wait()
        pend[b][1].wait()


def _sc_scatter(x_flat, pos_flat, wsrc):
    mesh = plsc.VectorSubcoreMesh(core_axis_name="c", subcore_axis_name="s")
    fn = pl.kernel(
        _sc_scatter_body, mesh=mesh,
        out_type=(jax.ShapeDtypeStruct((PROWS, C_DIM), jnp.float32),
                  jax.ShapeDtypeStruct((PROWS, 128), jnp.float32)),
        scratch_types=[pltpu.VMEM((SCH, C_DIM), jnp.float32),
                       pltpu.VMEM((SCH, C_DIM), jnp.float32),
                       pltpu.VMEM((SCH, 128), jnp.float32),
                       pltpu.VMEM((SCH, 128), jnp.float32),
                       pltpu.VMEM((SCH,), jnp.int32),
                       pltpu.VMEM((SCH,), jnp.int32),
                       pltpu.SemaphoreType.DMA,
                       pltpu.SemaphoreType.DMA,
                       pltpu.SemaphoreType.DMA,
                       pltpu.SemaphoreType.DMA],
    )
    return fn(x_flat, pos_flat, wsrc)


def _ffn1_body(meta_ref, xg_ref, w1_ref, b1_ref, h_ref):
    t = pl.program_id(0)
    used = meta_ref[31]

    @pl.when(t < used)
    def _():
        h = jnp.dot(xg_ref[...].astype(jnp.bfloat16), w1_ref[0].astype(jnp.bfloat16),
                    preferred_element_type=jnp.float32) + b1_ref[0]
        h = 0.5 * h * (1.0 + lax.erf(h * 0.7071067811865476))
        h_ref[...] = h.astype(jnp.bfloat16)


def _ffn2_body(meta_ref, h_ref, w2_ref, b2_ref, ws_ref, out_ref):
    t = pl.program_id(0)
    used = meta_ref[31]

    @pl.when(t < used)
    def _():
        contrib = jnp.dot(h_ref[...], w2_ref[0].astype(jnp.bfloat16),
                          preferred_element_type=jnp.float32)
        out_ref[...] = (contrib + b2_ref[0]) * ws_ref[:, 0:1]


def _ffn_call(meta, xg, W1, b1, W2, b2, wS):
    def live_in(t, m):
        return jnp.where(t < m[31], t, 0)

    def live_out(t, m):
        return jnp.where(t < m[31], t, MAXT)

    grid_spec1 = pltpu.PrefetchScalarGridSpec(
        num_scalar_prefetch=1,
        grid=(MAXT,),
        in_specs=[
            pl.BlockSpec((TILE, C_DIM), lambda t, m: (live_in(t, m), 0)),
            pl.BlockSpec((1, C_DIM, F_DIM), lambda t, m: (m[t], 0, 0)),
            pl.BlockSpec((1, 1, F_DIM), lambda t, m: (m[t], 0, 0)),
        ],
        out_specs=pl.BlockSpec((TILE, F_DIM), lambda t, m: (live_out(t, m), 0)),
    )
    h = pl.pallas_call(
        _ffn1_body, grid_spec=grid_spec1,
        out_shape=jax.ShapeDtypeStruct((PROWS + TILE, F_DIM), jnp.bfloat16),
        compiler_params=pltpu.CompilerParams(vmem_limit_bytes=60 * 1024 * 1024),
    )(meta, xg, W1, b1)
    grid_spec2 = pltpu.PrefetchScalarGridSpec(
        num_scalar_prefetch=1,
        grid=(MAXT,),
        in_specs=[
            pl.BlockSpec((TILE, F_DIM), lambda t, m: (live_in(t, m), 0)),
            pl.BlockSpec((1, F_DIM, C_DIM), lambda t, m: (m[t], 0, 0)),
            pl.BlockSpec((1, 1, C_DIM), lambda t, m: (m[t], 0, 0)),
            pl.BlockSpec((TILE, 128), lambda t, m: (live_in(t, m), 0)),
        ],
        out_specs=pl.BlockSpec((TILE, C_DIM), lambda t, m: (live_out(t, m), 0)),
    )
    return pl.pallas_call(
        _ffn2_body, grid_spec=grid_spec2,
        out_shape=jax.ShapeDtypeStruct((PROWS + TILE, C_DIM), jnp.float32),
        compiler_params=pltpu.CompilerParams(vmem_limit_bytes=60 * 1024 * 1024),
    )(meta, h, W2, b2, wS)


GCH = 32


def _sc_gather_body(o_hbm, pos_hbm, out_hbm, r0, r1, idx0, idx1, sem0, sem1):
    wid = lax.axis_index("s") * 2 + lax.axis_index("c")
    tb = wid * CH
    for sub in range(CH // GCH):
        cb = tb + sub * GCH
        pltpu.sync_copy(pos_hbm.at[pl.ds(cb, GCH)], idx0)
        pltpu.sync_copy(pos_hbm.at[pl.ds(N_TOK + cb, GCH)], idx1)
        c0 = pltpu.async_copy(o_hbm.at[idx0], r0, sem0)
        c1 = pltpu.async_copy(o_hbm.at[idx1], r1, sem1)
        c0.wait()
        c1.wait()

        def row_add(j, _):
            for c in range(C_DIM // 16):
                sl = (j, pl.ds(c * 16, 16))
                r0[sl] = r0[sl] + r1[sl]
            return 0

        lax.fori_loop(0, GCH, row_add, 0)
        pltpu.sync_copy(r0, out_hbm.at[pl.ds(cb, GCH)])


def _sc_gather_add(o, pos_flat):
    mesh = plsc.VectorSubcoreMesh(core_axis_name="c", subcore_axis_name="s")
    fn = pl.kernel(
        _sc_gather_body, mesh=mesh,
        out_type=jax.ShapeDtypeStruct((N_TOK, C_DIM), jnp.float32),
        scratch_types=[pltpu.VMEM((GCH, C_DIM), jnp.float32),
                       pltpu.VMEM((GCH, C_DIM), jnp.float32),
                       pltpu.VMEM((GCH,), jnp.int32),
                       pltpu.VMEM((GCH,), jnp.int32),
                       pltpu.SemaphoreType.DMA,
                       pltpu.SemaphoreType.DMA],
    )
    return fn(o, pos_flat)


def kernel(x, router_W, router_b, W1, b1, W2, b2):
    B, T, C = x.shape
    x_flat = x.reshape(T, C)
    pos, wts, meta, bal = _router_call(x_flat, router_W, router_b)
    pos_flat = pos.T.reshape(TOPK * N_TOK)
    wsrc = jnp.broadcast_to(wts.T.reshape(TOPK * N_TOK, 1), (TOPK * N_TOK, 128))
    xg, wS = _sc_scatter(x_flat, pos_flat, wsrc)
    o = _ffn_call(meta.reshape(32), xg,
                  W1, b1.reshape(N_EXP, 1, F_DIM),
                  W2, b2.reshape(N_EXP, 1, C_DIM), wS)
    out = _sc_gather_add(o, pos_flat)
    return out.reshape(B, T, C), bal.reshape(())


# speedup vs baseline: 1.3002x; 1.0340x over previous
import jax
import jax.numpy as jnp
from jax import lax
from jax.experimental import pallas as pl
from jax.experimental.pallas import tpu as pltpu
from jax.experimental.pallas import tpu_sc as plsc

N_TOK = 2048
C_DIM = 1024
N_EXP = 8
F_DIM = 4096
TOPK = 2
TILE = 576
MAXT = 14
PROWS = MAXT * TILE
FBLK = 1024
NFB = F_DIM // FBLK
CH = 64


def _router_body(x_ref, w_ref, b_ref, pos_ref, wts_ref, meta_ref, bal_ref):
    f32 = jnp.float32
    xv = x_ref[...]
    logits = jnp.dot(xv, w_ref[...], preferred_element_type=f32) + b_ref[...]
    m = jnp.max(logits, axis=1, keepdims=True)
    ex = jnp.exp(logits - m)
    probs = ex / jnp.sum(ex, axis=1, keepdims=True)
    mean_p = jnp.sum(probs, axis=0, keepdims=True) * (1.0 / N_TOK)
    bal_ref[...] = jnp.sum(mean_p * mean_p, axis=1, keepdims=True)
    lane = lax.broadcasted_iota(jnp.int32, (N_TOK, N_EXP), 1)
    i1 = jnp.min(jnp.where(logits == m, lane, N_EXP), axis=1, keepdims=True)
    masked = jnp.where(lane == i1, -jnp.inf, logits)
    m2 = jnp.max(masked, axis=1, keepdims=True)
    i2 = jnp.min(jnp.where(masked == m2, lane, N_EXP), axis=1, keepdims=True)
    v1 = jnp.sum(jnp.where(lane == i1, probs, 0.0), axis=1, keepdims=True)
    v2 = jnp.sum(jnp.where(lane == i2, probs, 0.0), axis=1, keepdims=True)
    wts_ref[:, 0:1] = v1
    wts_ref[:, 1:2] = v2
    I1 = (lane == i1).astype(f32)
    I2 = (lane == i2).astype(f32)
    counts = jnp.sum(I1 + I2, axis=0, keepdims=True)
    nt = jnp.floor((counts + (TILE - 1)) * (1.0 / TILE))
    er = lax.broadcasted_iota(jnp.int32, (N_EXP, N_EXP), 0)
    ec = lax.broadcasted_iota(jnp.int32, (N_EXP, N_EXP), 1)
    strict = (er < ec).astype(f32)
    tiles_before = jnp.dot(nt, strict, preferred_element_type=f32)
    start = tiles_before * float(TILE)
    tiles_incl = tiles_before + nt
    ident = (er == ec).astype(f32)
    ti_col = jnp.sum(ident * tiles_incl, axis=1, keepdims=True)
    lane32 = lax.broadcasted_iota(jnp.int32, (1, 32), 1).astype(f32)
    te = jnp.sum((ti_col <= lane32).astype(f32), axis=0, keepdims=True)
    te = jnp.minimum(te, float(N_EXP - 1))
    used = tiles_incl[:, N_EXP - 1:N_EXP]
    meta_ref[...] = jnp.where(lane32 == 31.0, used, te).astype(jnp.int32)
    L = (lax.broadcasted_iota(jnp.int32, (128, 128), 0)
         >= lax.broadcasted_iota(jnp.int32, (128, 128), 1)).astype(f32)
    lane8 = lax.broadcasted_iota(jnp.int32, (128, N_EXP), 1)
    carry = jnp.zeros((1, N_EXP), f32)
    for blk in range(32):
        k, rb = divmod(blk, 16)
        r0 = rb * 128
        ei = (i1 if k == 0 else i2)[r0:r0 + 128, :]
        Ic = (lane8 == ei).astype(f32)
        incl = jnp.dot(L, Ic, preferred_element_type=f32)
        rank = carry + incl - Ic
        posb = jnp.sum(Ic * (start + rank), axis=1, keepdims=True)
        pos_ref[r0:r0 + 128, k:k + 1] = posb.astype(jnp.int32)
        carry = carry + incl[127:128, :]


def _router_call(x_flat, router_W, router_b):
    return pl.pallas_call(
        _router_body,
        out_shape=(
            jax.ShapeDtypeStruct((N_TOK, TOPK), jnp.int32),
            jax.ShapeDtypeStruct((N_TOK, TOPK), jnp.float32),
            jax.ShapeDtypeStruct((1, 32), jnp.int32),
            jax.ShapeDtypeStruct((1, 1), jnp.float32),
        ),
    )(x_flat, router_W, router_b.reshape(1, N_EXP))


SCH = 32


def _sc_scatter_body(x_hbm, pos_hbm, wsrc_hbm, xg_hbm, ws_hbm,
                     xb0, xb1, wb0, wb1, ib0, ib1, s0, s1, t0, t1):
    wid = lax.axis_index("s") * 2 + lax.axis_index("c")
    k = wid // 16
    base = (wid % 16) * 128
    xb = (xb0, xb1)
    wb = (wb0, wb1)
    ib = (ib0, ib1)
    xsem = (s0, s1)
    wsem = (t0, t1)
    pend = [None, None]
    for c in range(4):
        b = c % 2
        if pend[b] is not None:
            pend[b][0].wait()
            pend[b][1].wait()
        tb = base + c * SCH
        pltpu.sync_copy(pos_hbm.at[pl.ds(k * N_TOK + tb, SCH)], ib[b])
        pltpu.sync_copy(wsrc_hbm.at[pl.ds(k * N_TOK + tb, SCH)], wb[b])
        pltpu.sync_copy(x_hbm.at[pl.ds(tb, SCH)], xb[b])
        pend[b] = (pltpu.async_copy(xb[b], xg_hbm.at[ib[b]], xsem[b]),
                   pltpu.async_copy(wb[b], ws_hbm.at[ib[b]], wsem[b]))
    for b in range(2):
        pend[b][0].wait()
        pend[b][1].wait()


def _sc_scatter(x_flat, pos_flat, wsrc):
    mesh = plsc.VectorSubcoreMesh(core_axis_name="c", subcore_axis_name="s")
    fn = pl.kernel(
        _sc_scatter_body, mesh=mesh,
        out_type=(jax.ShapeDtypeStruct((PROWS, C_DIM), jnp.float32),
                  jax.ShapeDtypeStruct((PROWS, 128), jnp.float32)),
        scratch_types=[pltpu.VMEM((SCH, C_DIM), jnp.float32),
                       pltpu.VMEM((SCH, C_DIM), jnp.float32),
                       pltpu.VMEM((SCH, 128), jnp.float32),
                       pltpu.VMEM((SCH, 128), jnp.float32),
                       pltpu.VMEM((SCH,), jnp.int32),
                       pltpu.VMEM((SCH,), jnp.int32),
                       pltpu.SemaphoreType.DMA,
                       pltpu.SemaphoreType.DMA,
                       pltpu.SemaphoreType.DMA,
                       pltpu.SemaphoreType.DMA],
    )
    return fn(x_flat, pos_flat, wsrc)


def _ffn1_body(meta_ref, xg_ref, w1_ref, b1_ref, h_ref):
    t = pl.program_id(0)
    used = meta_ref[31]

    @pl.when(t < used)
    def _():
        h = jnp.dot(xg_ref[...].astype(jnp.bfloat16), w1_ref[0].astype(jnp.bfloat16),
                    preferred_element_type=jnp.float32) + b1_ref[0]
        h = 0.5 * h * (1.0 + lax.erf(h * 0.7071067811865476))
        h_ref[...] = h.astype(jnp.bfloat16)


def _ffn2_body(meta_ref, h_ref, w2_ref, b2_ref, ws_ref, out_ref):
    t = pl.program_id(0)
    used = meta_ref[31]

    @pl.when(t < used)
    def _():
        contrib = jnp.dot(h_ref[...], w2_ref[0].astype(jnp.bfloat16),
                          preferred_element_type=jnp.float32)
        out_ref[...] = (contrib + b2_ref[0]) * ws_ref[:, 0:1]


def _ffn_call(meta, xg, W1, b1, W2, b2, wS):
    def live_in(t, m):
        return jnp.where(t < m[31], t, 0)

    def live_out(t, m):
        return jnp.where(t < m[31], t, MAXT)

    grid_spec1 = pltpu.PrefetchScalarGridSpec(
        num_scalar_prefetch=1,
        grid=(MAXT,),
        in_specs=[
            pl.BlockSpec((TILE, C_DIM), lambda t, m: (live_in(t, m), 0)),
            pl.BlockSpec((1, C_DIM, F_DIM), lambda t, m: (m[t], 0, 0)),
            pl.BlockSpec((1, 1, F_DIM), lambda t, m: (m[t], 0, 0)),
        ],
        out_specs=pl.BlockSpec((TILE, F_DIM), lambda t, m: (live_out(t, m), 0)),
    )
    h = pl.pallas_call(
        _ffn1_body, grid_spec=grid_spec1,
        out_shape=jax.ShapeDtypeStruct((PROWS + TILE, F_DIM), jnp.bfloat16),
        compiler_params=pltpu.CompilerParams(vmem_limit_bytes=60 * 1024 * 1024),
    )(meta, xg, W1, b1)
    grid_spec2 = pltpu.PrefetchScalarGridSpec(
        num_scalar_prefetch=1,
        grid=(MAXT,),
        in_specs=[
            pl.BlockSpec((TILE, F_DIM), lambda t, m: (live_in(t, m), 0)),
            pl.BlockSpec((1, F_DIM, C_DIM), lambda t, m: (m[t], 0, 0)),
            pl.BlockSpec((1, 1, C_DIM), lambda t, m: (m[t], 0, 0)),
            pl.BlockSpec((TILE, 128), lambda t, m: (live_in(t, m), 0)),
        ],
        out_specs=pl.BlockSpec((TILE, C_DIM), lambda t, m: (live_out(t, m), 0)),
    )
    return pl.pallas_call(
        _ffn2_body, grid_spec=grid_spec2,
        out_shape=jax.ShapeDtypeStruct((PROWS + TILE, C_DIM), jnp.float32),
        compiler_params=pltpu.CompilerParams(vmem_limit_bytes=60 * 1024 * 1024),
    )(meta, h, W2, b2, wS)


GCH = 32


def _sc_gather_body(o_hbm, pos_hbm, out_hbm, r0, r1, idx0, idx1, sem0, sem1):
    wid = lax.axis_index("s") * 2 + lax.axis_index("c")
    tb = wid * CH
    for sub in range(CH // GCH):
        cb = tb + sub * GCH
        pltpu.sync_copy(pos_hbm.at[pl.ds(cb, GCH)], idx0)
        pltpu.sync_copy(pos_hbm.at[pl.ds(N_TOK + cb, GCH)], idx1)
        c0 = pltpu.async_copy(o_hbm.at[idx0], r0, sem0)
        c1 = pltpu.async_copy(o_hbm.at[idx1], r1, sem1)
        c0.wait()
        c1.wait()

        def row_add(j, _):
            for c in range(C_DIM // 16):
                sl = (j, pl.ds(c * 16, 16))
                r0[sl] = r0[sl] + r1[sl]
            return 0

        lax.fori_loop(0, GCH, row_add, 0)
        pltpu.sync_copy(r0, out_hbm.at[pl.ds(cb, GCH)])


def _sc_gather_add(o, pos_flat):
    mesh = plsc.VectorSubcoreMesh(core_axis_name="c", subcore_axis_name="s")
    fn = pl.kernel(
        _sc_gather_body, mesh=mesh,
        out_type=jax.ShapeDtypeStruct((N_TOK, C_DIM), jnp.float32),
        scratch_types=[pltpu.VMEM((GCH, C_DIM), jnp.float32),
                       pltpu.VMEM((GCH, C_DIM), jnp.float32),
                       pltpu.VMEM((GCH,), jnp.int32),
                       pltpu.VMEM((GCH,), jnp.int32),
                       pltpu.SemaphoreType.DMA,
                       pltpu.SemaphoreType.DMA],
    )
    return fn(o, pos_flat)


def kernel(x, router_W, router_b, W1, b1, W2, b2):
    B, T, C = x.shape
    x_flat = x.reshape(T, C)
    pos, wts, meta, bal = _router_call(x_flat, router_W, router_b)
    pos_flat = pos.T.reshape(TOPK * N_TOK)
    wsrc = jnp.broadcast_to(wts.T.reshape(TOPK * N_TOK, 1), (TOPK * N_TOK, 128))
    xg, wS = _sc_scatter(x_flat, pos_flat, wsrc)
    o = _ffn_call(meta.reshape(32), xg,
                  W1, b1.reshape(N_EXP, 1, F_DIM),
                  W2, b2.reshape(N_EXP, 1, C_DIM), wS)
    out = _sc_gather_add(o, pos_flat)
    return out.reshape(B, T, C), bal.reshape(())
